# baseline (speedup 1.0000x reference)
import functools

import numpy as np
import jax
import jax.numpy as jnp
from jax import lax
from jax.experimental import pallas as pl
from jax.experimental.pallas import tpu as pltpu
from jax.experimental.pallas import tpu_sc as plsc

NUM_BASIS = 10
MUL = 16
N_NODES = 10000
N_EDGES = 160000
OUT_DIM = 144
NC, NS = 2, 16
NW = NC * NS
CH = 128
ROWS = N_EDGES // CH
RPC = ROWS // NC
NPT = N_NODES // NS
BE = 3200
BROWS = N_NODES * MUL // CH


def _static_mats():
    l = np.zeros(OUT_DIM, np.int32)
    v = np.zeros(OUT_DIM, np.int32)
    j = np.zeros(OUT_DIM, np.int32)
    for c in range(OUT_DIM):
        if c < 16:
            l[c], v[c], j[c] = 0, c, 0
        elif c < 64:
            l[c], v[c], j[c] = 1, (c - 16) // 3, (c - 16) % 3
        else:
            l[c], v[c], j[c] = 2, (c - 64) // 5, (c - 64) % 5
    colmap = l * MUL + v
    RH = np.zeros((MUL, MUL * MUL), np.float32)
    RX = np.zeros((MUL, MUL * MUL), np.float32)
    for k in range(MUL):
        for u in range(MUL):
            RH[k, k * MUL + u] = 1.0
            RX[u, k * MUL + u] = 1.0
    amap = [3, 0, 1, 2, 0, 0, 1, 1, 2, 0]
    bmap = [3, 3, 3, 3, 2, 1, 1, 2, 2, 0]
    A = np.zeros((8, 16), np.float32)
    B = np.zeros((8, 16), np.float32)
    for t in range(10):
        A[amap[t], t] = 1.0
        B[bmap[t], t] = 1.0
    r3, c2, r5 = np.sqrt(3.0), np.sqrt(15.0), np.sqrt(5.0)
    C = np.zeros((16, 9), np.float32)
    C[0, 0] = 1.0
    C[1, 1] = r3
    C[2, 2] = r3
    C[3, 3] = r3
    C[4, 4] = c2
    C[5, 5] = c2
    C[6, 6] = 3.0 * r5 / 2.0
    C[0, 6] = -r5 / 2.0
    C[7, 7] = c2
    C[8, 8] = c2 / 2.0
    C[9, 8] = -c2 / 2.0
    jg = np.where(l == 0, 0, np.where(l == 1, 1 + j, 4 + j))
    Q = np.zeros((9, OUT_DIM), np.float32)
    for c in range(OUT_DIM):
        Q[jg[c], c] = 1.0
    CQ = C @ Q
    return colmap, RH, RX, A, B, CQ


_COLMAP, _RH, _RX, _QA, _QB, _CQ = _static_mats()


def _sc_gather(t1, t2, src2d, dst2d):
    mesh = plsc.VectorSubcoreMesh(core_axis_name="c", subcore_axis_name="s")

    @functools.partial(
        pl.kernel,
        out_type=(jax.ShapeDtypeStruct((N_EDGES, 32), jnp.float32),
                  jax.ShapeDtypeStruct((N_EDGES, 16), jnp.float32)),
        mesh=mesh,
        scratch_types=[
            pltpu.VMEM((CH,), jnp.int32),
            pltpu.VMEM((CH,), jnp.int32),
            pltpu.VMEM((CH, 32), jnp.float32),
            pltpu.VMEM((CH, 16), jnp.float32),
            pltpu.SemaphoreType.DMA,
            pltpu.SemaphoreType.DMA,
        ],
        compiler_params=pltpu.CompilerParams(use_tc_tiling_on_sc=False),
    )
    def k(t1h, t2h, srch, dsth, out1, out2, idx1, idx2, buf1, buf2, sem1, sem2):
        w = lax.axis_index("s") * NC + lax.axis_index("c")

        def body(i, carry):
            row = w + i * NW

            @pl.when(row < ROWS)
            def _():
                pltpu.sync_copy(srch.at[row], idx1)
                pltpu.sync_copy(dsth.at[row], idx2)
                cp1 = pltpu.async_copy(t1h.at[idx1], buf1, sem1)
                cp2 = pltpu.async_copy(t2h.at[idx2], buf2, sem2)
                cp1.wait()
                cp2.wait()
                pltpu.sync_copy(buf1, out1.at[pl.ds(row * CH, CH)])
                pltpu.sync_copy(buf2, out2.at[pl.ds(row * CH, CH)])

            return carry

        lax.fori_loop(0, (ROWS + NW - 1) // NW, body, 0)

    return k(t1, t2, src2d, dst2d)


def _tc_msg(srcg4, dstg8, w1p, w2p, rh, rx, qa, qb, cq, prm):
    grid = N_EDGES // BE

    def body(prm_ref, w1_ref, w2_ref, rh_ref, rx_ref, qa_ref, qb_ref, cq_ref,
             s_ref, d_ref, oa_ref, ob_ref):
        sg = s_ref[...]
        dg = d_ref[...]
        x = sg[:, 0:16]
        ev = dg[:, 0:8] - sg[:, 16:24]
        sq = ev * ev
        hi = lax.Precision.HIGHEST
        d2_16 = jnp.dot(sq, jnp.ones((8, 16), jnp.float32),
                        preferred_element_type=jnp.float32, precision=hi)
        dist16 = jnp.sqrt(d2_16 + 1e-9)
        rinv16 = 1.0 / dist16
        u9 = ev * rinv16[:, 0:8] + prm_ref[2:3, 0:8]
        quad = (jnp.dot(u9, qa_ref[...], preferred_element_type=jnp.float32,
                        precision=hi)
                * jnp.dot(u9, qb_ref[...], preferred_element_type=jnp.float32,
                          precision=hi))
        s_sh = jnp.dot(quad, cq_ref[...], preferred_element_type=jnp.float32,
                       precision=hi)
        diff = (dist16 - prm_ref[0:1, :]) * prm_ref[1:2, :]
        t2 = diff * diff
        den = 1.0 - t2
        arg = -2.0 / den
        soft = (1.14136 * np.exp(2.0)) * jnp.exp(arg)
        valid = (t2 < 1.0) & (lax.broadcasted_iota(jnp.int32, (BE, 16), 1)
                              < NUM_BASIS)
        soft = jnp.where(valid, soft, 0.0)
        h = jax.nn.relu(jnp.dot(soft, w1_ref[...],
                                preferred_element_type=jnp.float32))
        g = (jnp.dot(h, rh_ref[...], preferred_element_type=jnp.float32,
                     precision=hi)
             * jnp.dot(x, rx_ref[...], preferred_element_type=jnp.float32,
                       precision=hi))
        m = jnp.dot(g, w2_ref[...], preferred_element_type=jnp.float32)
        msg = m * s_sh
        oa_ref[...] = msg[:, 0:128]
        ob_ref[...] = msg[:, 128:144]

    small = lambda shp: pl.BlockSpec(shp, lambda i: (0, 0))
    return pl.pallas_call(
        body,
        grid=(grid,),
        in_specs=[
            small((8, 16)),
            small((16, 16)),
            small((256, OUT_DIM)),
            small((16, 256)),
            small((16, 256)),
            small((8, 16)),
            small((8, 16)),
            small((16, OUT_DIM)),
            pl.BlockSpec((BE, 32), lambda i: (i, 0)),
            pl.BlockSpec((BE, 16), lambda i: (i, 0)),
        ],
        out_specs=(pl.BlockSpec((BE, 128), lambda i: (i, 0)),
                   pl.BlockSpec((BE, 16), lambda i: (i, 0))),
        out_shape=(jax.ShapeDtypeStruct((N_EDGES, 128), jnp.float32),
                   jax.ShapeDtypeStruct((N_EDGES, 16), jnp.float32)),
    )(prm, w1p, w2p, rh, rx, qa, qb, cq, srcg4, dstg8)


def _sc_scatter(msga, msgb, dst2d):
    mesh = plsc.VectorSubcoreMesh(core_axis_name="c", subcore_axis_name="s")

    @functools.partial(
        pl.kernel,
        out_type=(jax.ShapeDtypeStruct((NC * N_NODES, 128), jnp.float32),
                  jax.ShapeDtypeStruct((NC * N_NODES, 16), jnp.float32)),
        mesh=mesh,
        scratch_types=[
            pltpu.VMEM((1, CH), jnp.int32),
            pltpu.VMEM((CH, 128), jnp.float32),
            pltpu.VMEM((CH, 16), jnp.float32),
            pltpu.VMEM_SHARED((N_NODES, 128), jnp.float32),
            pltpu.VMEM_SHARED((N_NODES, 16), jnp.float32),
        ],
        compiler_params=pltpu.CompilerParams(use_tc_tiling_on_sc=False),
    )
    def k(mah, mbh, dsth, outa, outb, idxb, bufa, bufb, acca, accb):
        c = lax.axis_index("c")
        s = lax.axis_index("s")
        zeros16 = jnp.zeros((16,), jnp.float32)

        def zrow(i, carry):
            def zcol(t, carry2):
                bufa[i, pl.ds(t * 16, 16)] = zeros16
                return carry2

            lax.fori_loop(0, 8, zcol, carry)
            bufb[i, :] = zeros16
            return carry

        lax.fori_loop(0, CH, zrow, 0)

        def zcp(t, carry):
            pltpu.sync_copy(bufa, acca.at[pl.ds(s * NPT + t * CH, CH)])
            pltpu.sync_copy(bufb, accb.at[pl.ds(s * NPT + t * CH, CH)])
            return carry

        lax.fori_loop(0, NPT // CH, zcp, 0)
        rem = NPT - (NPT // CH) * CH
        pltpu.sync_copy(bufa.at[pl.ds(0, rem)],
                        acca.at[pl.ds(s * NPT + NPT - rem, rem)])
        pltpu.sync_copy(bufb.at[pl.ds(0, rem)],
                        accb.at[pl.ds(s * NPT + NPT - rem, rem)])
        plsc.subcore_barrier()

        def body(i, carry):
            lrow = s + i * NS

            @pl.when(lrow < RPC)
            def _():
                row = c * RPC + lrow
                pltpu.sync_copy(dsth.at[row], idxb.at[0])
                pltpu.sync_copy(mah.at[pl.ds(row * CH, CH)], bufa)
                pltpu.sync_copy(mbh.at[pl.ds(row * CH, CH)], bufb)
                pltpu.sync_copy(bufa, acca.at[idxb.at[0]], add=True)
                pltpu.sync_copy(bufb, accb.at[idxb.at[0]], add=True)

            return carry

        lax.fori_loop(0, (RPC + NS - 1) // NS, body, 0)
        plsc.subcore_barrier()
        pltpu.sync_copy(acca.at[pl.ds(s * NPT, NPT)],
                        outa.at[pl.ds(c * N_NODES + s * NPT, NPT)])
        pltpu.sync_copy(accb.at[pl.ds(s * NPT, NPT)],
                        outb.at[pl.ds(c * N_NODES + s * NPT, NPT)])

    return k(msga, msgb, dst2d)


def _tc_add(parta, partb):
    def body(a_ref, b_ref, o_ref):
        o_ref[:, 0:128] = a_ref[0:N_NODES, :] + a_ref[N_NODES:2 * N_NODES, :]
        o_ref[:, 128:144] = b_ref[0:N_NODES, :] + b_ref[N_NODES:2 * N_NODES, :]

    return pl.pallas_call(
        body,
        out_shape=jax.ShapeDtypeStruct((N_NODES, OUT_DIM), jnp.float32),
    )(parta, partb)


def kernel(f_in, pos, edge_src, edge_dst, max_radius, W1, W2):
    f_in = f_in.astype(jnp.float32)
    pos = pos.astype(jnp.float32)
    t1 = jnp.concatenate([f_in, pos, jnp.zeros((N_NODES, 13), jnp.float32)], axis=1)
    t2 = jnp.concatenate([pos, jnp.zeros((N_NODES, 13), jnp.float32)], axis=1)
    src2d = edge_src.astype(jnp.int32).reshape(ROWS, CH)
    dst2d = edge_dst.astype(jnp.int32).reshape(ROWS, CH)

    w2r = W2.astype(jnp.float32).reshape(MUL, 3, MUL, MUL)
    w2kl = jnp.transpose(w2r, (0, 2, 1, 3)).reshape(MUL * MUL, 3 * MUL)
    w2p = jnp.take(w2kl, jnp.asarray(_COLMAP), axis=1) * (1.0 / 64.0)
    w1p = (jnp.zeros((16, 16), jnp.float32)
           .at[:NUM_BASIS].set(W1.astype(jnp.float32)) * np.sqrt(2.0))

    mr = jnp.asarray(max_radius, jnp.float32)
    step = mr / (NUM_BASIS + 1)
    vals = step * jnp.arange(1, NUM_BASIS + 1, dtype=jnp.float32)
    prm = (jnp.zeros((8, 16), jnp.float32)
           .at[0, :NUM_BASIS].set(vals)
           .at[1, :].set(1.0 / step)
           .at[2, 3].set(1.0))

    srcg, dstg = _sc_gather(t1, t2, src2d, dst2d)
    msga, msgb = _tc_msg(srcg, dstg, w1p, w2p,
                         jnp.asarray(_RH), jnp.asarray(_RX),
                         jnp.asarray(_QA), jnp.asarray(_QB),
                         jnp.asarray(_CQ), prm)
    parta, partb = _sc_scatter(msga, msgb, dst2d)
    return _tc_add(parta, partb)


# speedup vs baseline: 1.6215x; 1.6215x over previous
import functools

import numpy as np
import jax
import jax.numpy as jnp
from jax import lax
from jax.experimental import pallas as pl
from jax.experimental.pallas import tpu as pltpu
from jax.experimental.pallas import tpu_sc as plsc

NUM_BASIS = 10
MUL = 16
N_NODES = 10000
N_EDGES = 160000
OUT_DIM = 144
NC, NS = 2, 16
NW = NC * NS
CH = 128
ROWS = N_EDGES // CH
RPC = ROWS // NC
NPT = N_NODES // NS
BE = 3200
BROWS = N_NODES * MUL // CH


def _static_mats():
    l = np.zeros(OUT_DIM, np.int32)
    v = np.zeros(OUT_DIM, np.int32)
    j = np.zeros(OUT_DIM, np.int32)
    for c in range(OUT_DIM):
        if c < 16:
            l[c], v[c], j[c] = 0, c, 0
        elif c < 64:
            l[c], v[c], j[c] = 1, (c - 16) // 3, (c - 16) % 3
        else:
            l[c], v[c], j[c] = 2, (c - 64) // 5, (c - 64) % 5
    colmap = l * MUL + v
    RH = np.zeros((MUL, MUL * MUL), np.float32)
    RX = np.zeros((MUL, MUL * MUL), np.float32)
    for k in range(MUL):
        for u in range(MUL):
            RH[k, k * MUL + u] = 1.0
            RX[u, k * MUL + u] = 1.0
    amap = [3, 0, 1, 2, 0, 0, 1, 1, 2, 0]
    bmap = [3, 3, 3, 3, 2, 1, 1, 2, 2, 0]
    A = np.zeros((8, 16), np.float32)
    B = np.zeros((8, 16), np.float32)
    for t in range(10):
        A[amap[t], t] = 1.0
        B[bmap[t], t] = 1.0
    r3, c2, r5 = np.sqrt(3.0), np.sqrt(15.0), np.sqrt(5.0)
    C = np.zeros((16, 9), np.float32)
    C[0, 0] = 1.0
    C[1, 1] = 1.0
    C[2, 2] = 1.0
    C[3, 3] = 1.0
    C[4, 4] = 1.0
    C[5, 5] = 1.0
    C[6, 6] = 3.0
    C[0, 6] = -1.0
    C[7, 7] = 1.0
    C[8, 8] = 1.0
    C[9, 8] = -1.0
    shconst = np.array([1.0, r3, r3, r3, c2, c2, r5 / 2.0, c2, c2 / 2.0],
                       np.float32)
    jg = np.where(l == 0, 0, np.where(l == 1, 1 + j, 4 + j))
    Q = np.zeros((9, OUT_DIM), np.float32)
    for c in range(OUT_DIM):
        Q[jg[c], c] = 1.0
    CQ = C @ Q
    colconst = shconst[jg]
    return colmap, colconst, RH, RX, A, B, CQ


_COLMAP, _COLCONST, _RH, _RX, _QA, _QB, _CQ = _static_mats()


def _split_bf16(x):
    hix = x.astype(jnp.bfloat16)
    lox = (x - hix.astype(jnp.float32)).astype(jnp.bfloat16)
    return hix, lox


def _route(x, sel):
    hix, lox = _split_bf16(x)
    return (jnp.dot(hix, sel, preferred_element_type=jnp.float32)
            + jnp.dot(lox, sel, preferred_element_type=jnp.float32))


def _sc_gather(t1, t2, src2d, dst2d):
    mesh = plsc.VectorSubcoreMesh(core_axis_name="c", subcore_axis_name="s")

    @functools.partial(
        pl.kernel,
        out_type=(jax.ShapeDtypeStruct((N_EDGES, 32), jnp.float32),
                  jax.ShapeDtypeStruct((N_EDGES, 16), jnp.float32)),
        mesh=mesh,
        scratch_types=[
            pltpu.VMEM((CH,), jnp.int32),
            pltpu.VMEM((CH,), jnp.int32),
            pltpu.VMEM((CH, 32), jnp.float32),
            pltpu.VMEM((CH, 16), jnp.float32),
            pltpu.SemaphoreType.DMA,
            pltpu.SemaphoreType.DMA,
        ],
        compiler_params=pltpu.CompilerParams(use_tc_tiling_on_sc=False),
    )
    def k(t1h, t2h, srch, dsth, out1, out2, idx1, idx2, buf1, buf2, sem1, sem2):
        w = lax.axis_index("s") * NC + lax.axis_index("c")

        def body(i, carry):
            row = w + i * NW

            @pl.when(row < ROWS)
            def _():
                pltpu.sync_copy(srch.at[row], idx1)
                pltpu.sync_copy(dsth.at[row], idx2)
                cp1 = pltpu.async_copy(t1h.at[idx1], buf1, sem1)
                cp2 = pltpu.async_copy(t2h.at[idx2], buf2, sem2)
                cp1.wait()
                cp2.wait()
                pltpu.sync_copy(buf1, out1.at[pl.ds(row * CH, CH)])
                pltpu.sync_copy(buf2, out2.at[pl.ds(row * CH, CH)])

            return carry

        lax.fori_loop(0, (ROWS + NW - 1) // NW, body, 0)

    return k(t1, t2, src2d, dst2d)


def _tc_msg(srcg4, dstg8, w1p, w2p, rh, rx, qa, qb, cq, prm):
    grid = N_EDGES // BE

    def body(prm_ref, w1_ref, w2_ref, rh_ref, rx_ref, qa_ref, qb_ref, cq_ref,
             s_ref, d_ref, oa_ref, ob_ref):
        sg = s_ref[...]
        dg = d_ref[...]
        x = sg[:, 0:16]
        ev = dg[:, 0:8] - sg[:, 16:24]
        sq = ev * ev
        d2 = jnp.sum(sq, axis=1, keepdims=True)
        dist = jnp.sqrt(d2 + 1e-9)
        rinv = 1.0 / dist
        u9 = ev * rinv + prm_ref[2:3, 0:8]
        quad = _route(u9, qa_ref[...]) * _route(u9, qb_ref[...])
        s_sh = _route(quad, cq_ref[...])
        diff = (dist - prm_ref[0:1, :]) * prm_ref[1:2, :]
        t2 = diff * diff
        den = 1.0 - t2
        arg = -2.0 / den
        soft = (1.14136 * np.exp(2.0)) * jnp.exp(arg)
        valid = (t2 < 1.0) & (lax.broadcasted_iota(jnp.int32, (BE, 16), 1)
                              < NUM_BASIS)
        soft = jnp.where(valid, soft, 0.0)
        h = jax.nn.relu(jnp.dot(soft, w1_ref[...],
                                preferred_element_type=jnp.float32))
        g = (jnp.dot(h, rh_ref[...], preferred_element_type=jnp.float32)
             * jnp.dot(x, rx_ref[...], preferred_element_type=jnp.float32))
        m = jnp.dot(g, w2_ref[...], preferred_element_type=jnp.float32)
        msg = m * s_sh
        oa_ref[...] = msg[:, 0:128]
        ob_ref[...] = msg[:, 128:144]

    small = lambda shp: pl.BlockSpec(shp, lambda i: (0, 0))
    return pl.pallas_call(
        body,
        grid=(grid,),
        in_specs=[
            small((8, 16)),
            small((16, 16)),
            small((256, OUT_DIM)),
            small((16, 256)),
            small((16, 256)),
            small((8, 16)),
            small((8, 16)),
            small((16, OUT_DIM)),
            pl.BlockSpec((BE, 32), lambda i: (i, 0)),
            pl.BlockSpec((BE, 16), lambda i: (i, 0)),
        ],
        out_specs=(pl.BlockSpec((BE, 128), lambda i: (i, 0)),
                   pl.BlockSpec((BE, 16), lambda i: (i, 0))),
        out_shape=(jax.ShapeDtypeStruct((N_EDGES, 128), jnp.float32),
                   jax.ShapeDtypeStruct((N_EDGES, 16), jnp.float32)),
    )(prm, w1p, w2p, rh, rx, qa, qb, cq, srcg4, dstg8)


def _sc_scatter(msga, msgb, dst2d):
    mesh = plsc.VectorSubcoreMesh(core_axis_name="c", subcore_axis_name="s")

    @functools.partial(
        pl.kernel,
        out_type=(jax.ShapeDtypeStruct((NC * N_NODES, 128), jnp.float32),
                  jax.ShapeDtypeStruct((NC * N_NODES, 16), jnp.float32)),
        mesh=mesh,
        scratch_types=[
            pltpu.VMEM((1, CH), jnp.int32),
            pltpu.VMEM((CH, 128), jnp.float32),
            pltpu.VMEM((CH, 16), jnp.float32),
            pltpu.VMEM_SHARED((N_NODES, 128), jnp.float32),
            pltpu.VMEM_SHARED((N_NODES, 16), jnp.float32),
        ],
        compiler_params=pltpu.CompilerParams(use_tc_tiling_on_sc=False),
    )
    def k(mah, mbh, dsth, outa, outb, idxb, bufa, bufb, acca, accb):
        c = lax.axis_index("c")
        s = lax.axis_index("s")
        zeros16 = jnp.zeros((16,), jnp.float32)

        def zrow(i, carry):
            def zcol(t, carry2):
                bufa[i, pl.ds(t * 16, 16)] = zeros16
                return carry2

            lax.fori_loop(0, 8, zcol, carry)
            bufb[i, :] = zeros16
            return carry

        lax.fori_loop(0, CH, zrow, 0)

        def zcp(t, carry):
            pltpu.sync_copy(bufa, acca.at[pl.ds(s * NPT + t * CH, CH)])
            pltpu.sync_copy(bufb, accb.at[pl.ds(s * NPT + t * CH, CH)])
            return carry

        lax.fori_loop(0, NPT // CH, zcp, 0)
        rem = NPT - (NPT // CH) * CH
        pltpu.sync_copy(bufa.at[pl.ds(0, rem)],
                        acca.at[pl.ds(s * NPT + NPT - rem, rem)])
        pltpu.sync_copy(bufb.at[pl.ds(0, rem)],
                        accb.at[pl.ds(s * NPT + NPT - rem, rem)])
        plsc.subcore_barrier()

        def body(i, carry):
            lrow = s + i * NS

            @pl.when(lrow < RPC)
            def _():
                row = c * RPC + lrow
                pltpu.sync_copy(dsth.at[row], idxb.at[0])
                pltpu.sync_copy(mah.at[pl.ds(row * CH, CH)], bufa)
                pltpu.sync_copy(mbh.at[pl.ds(row * CH, CH)], bufb)
                pltpu.sync_copy(bufa, acca.at[idxb.at[0]], add=True)
                pltpu.sync_copy(bufb, accb.at[idxb.at[0]], add=True)

            return carry

        lax.fori_loop(0, (RPC + NS - 1) // NS, body, 0)
        plsc.subcore_barrier()
        pltpu.sync_copy(acca.at[pl.ds(s * NPT, NPT)],
                        outa.at[pl.ds(c * N_NODES + s * NPT, NPT)])
        pltpu.sync_copy(accb.at[pl.ds(s * NPT, NPT)],
                        outb.at[pl.ds(c * N_NODES + s * NPT, NPT)])

    return k(msga, msgb, dst2d)


def _tc_add(parta, partb):
    def body(a_ref, b_ref, o_ref):
        o_ref[:, 0:128] = a_ref[0:N_NODES, :] + a_ref[N_NODES:2 * N_NODES, :]
        o_ref[:, 128:144] = b_ref[0:N_NODES, :] + b_ref[N_NODES:2 * N_NODES, :]

    return pl.pallas_call(
        body,
        out_shape=jax.ShapeDtypeStruct((N_NODES, OUT_DIM), jnp.float32),
    )(parta, partb)


def kernel(f_in, pos, edge_src, edge_dst, max_radius, W1, W2):
    f_in = f_in.astype(jnp.float32)
    pos = pos.astype(jnp.float32)
    t1 = jnp.concatenate([f_in, pos, jnp.zeros((N_NODES, 13), jnp.float32)], axis=1)
    t2 = jnp.concatenate([pos, jnp.zeros((N_NODES, 13), jnp.float32)], axis=1)
    src2d = edge_src.astype(jnp.int32).reshape(ROWS, CH)
    dst2d = edge_dst.astype(jnp.int32).reshape(ROWS, CH)

    w2r = W2.astype(jnp.float32).reshape(MUL, 3, MUL, MUL)
    w2kl = jnp.transpose(w2r, (0, 2, 1, 3)).reshape(MUL * MUL, 3 * MUL)
    w2p = (jnp.take(w2kl, jnp.asarray(_COLMAP), axis=1) * (1.0 / 64.0)
           * jnp.asarray(_COLCONST)[None, :])
    w1p = (jnp.zeros((16, 16), jnp.float32)
           .at[:NUM_BASIS].set(W1.astype(jnp.float32)) * np.sqrt(2.0))

    mr = jnp.asarray(max_radius, jnp.float32)
    step = mr / (NUM_BASIS + 1)
    vals = step * jnp.arange(1, NUM_BASIS + 1, dtype=jnp.float32)
    prm = (jnp.zeros((8, 16), jnp.float32)
           .at[0, :NUM_BASIS].set(vals)
           .at[1, :].set(1.0 / step)
           .at[2, 3].set(1.0))

    srcg, dstg = _sc_gather(t1, t2, src2d, dst2d)
    msga, msgb = _tc_msg(srcg, dstg, w1p, w2p,
                         jnp.asarray(_RH), jnp.asarray(_RX),
                         jnp.asarray(_QA, dtype=jnp.bfloat16),
                         jnp.asarray(_QB, dtype=jnp.bfloat16),
                         jnp.asarray(_CQ, dtype=jnp.bfloat16), prm)
    parta, partb = _sc_scatter(msga, msgb, dst2d)
    return _tc_add(parta, partb)


# speedup vs baseline: 1.7044x; 1.0511x over previous
import functools

import numpy as np
import jax
import jax.numpy as jnp
from jax import lax
from jax.experimental import pallas as pl
from jax.experimental.pallas import tpu as pltpu
from jax.experimental.pallas import tpu_sc as plsc

NUM_BASIS = 10
MUL = 16
N_NODES = 10000
N_EDGES = 160000
OUT_DIM = 144
NC, NS = 2, 16
NW = NC * NS
CH = 128
ROWS = N_EDGES // CH
RPC = ROWS // NC
NPT = N_NODES // NS
BE = 3200
BROWS = N_NODES * MUL // CH


def _static_mats():
    l = np.zeros(OUT_DIM, np.int32)
    v = np.zeros(OUT_DIM, np.int32)
    j = np.zeros(OUT_DIM, np.int32)
    for c in range(OUT_DIM):
        if c < 16:
            l[c], v[c], j[c] = 0, c, 0
        elif c < 64:
            l[c], v[c], j[c] = 1, (c - 16) // 3, (c - 16) % 3
        else:
            l[c], v[c], j[c] = 2, (c - 64) // 5, (c - 64) % 5
    colmap = l * MUL + v
    RH = np.zeros((MUL, MUL * MUL), np.float32)
    RX = np.zeros((MUL, MUL * MUL), np.float32)
    for k in range(MUL):
        for u in range(MUL):
            RH[k, k * MUL + u] = 1.0
            RX[u, k * MUL + u] = 1.0
    amap = [3, 0, 1, 2, 0, 0, 1, 1, 2, 0]
    bmap = [3, 3, 3, 3, 2, 1, 1, 2, 2, 0]
    A = np.zeros((8, 16), np.float32)
    B = np.zeros((8, 16), np.float32)
    for t in range(10):
        A[amap[t], t] = 1.0
        B[bmap[t], t] = 1.0
    r3, c2, r5 = np.sqrt(3.0), np.sqrt(15.0), np.sqrt(5.0)
    C = np.zeros((16, 9), np.float32)
    C[0, 0] = 1.0
    C[1, 1] = 1.0
    C[2, 2] = 1.0
    C[3, 3] = 1.0
    C[4, 4] = 1.0
    C[5, 5] = 1.0
    C[6, 6] = 3.0
    C[0, 6] = -1.0
    C[7, 7] = 1.0
    C[8, 8] = 1.0
    C[9, 8] = -1.0
    shconst = np.array([1.0, r3, r3, r3, c2, c2, r5 / 2.0, c2, c2 / 2.0],
                       np.float32)
    jg = np.where(l == 0, 0, np.where(l == 1, 1 + j, 4 + j))
    Q = np.zeros((9, OUT_DIM), np.float32)
    for c in range(OUT_DIM):
        Q[jg[c], c] = 1.0
    CQ = C @ Q
    colconst = shconst[jg]
    return colmap, colconst, RH, RX, A, B, CQ


_COLMAP, _COLCONST, _RH, _RX, _QA, _QB, _CQ = _static_mats()


def _split_bf16(x):
    hix = x.astype(jnp.bfloat16)
    lox = (x - hix.astype(jnp.float32)).astype(jnp.bfloat16)
    return hix, lox


def _route(x, sel):
    hix, lox = _split_bf16(x)
    return (jnp.dot(hix, sel, preferred_element_type=jnp.float32)
            + jnp.dot(lox, sel, preferred_element_type=jnp.float32))


def _sc_gather(t1, t2, src2d, dst2d):
    mesh = plsc.VectorSubcoreMesh(core_axis_name="c", subcore_axis_name="s")

    @functools.partial(
        pl.kernel,
        out_type=jax.ShapeDtypeStruct((N_EDGES, 128), jnp.float32),
        mesh=mesh,
        scratch_types=[
            pltpu.VMEM((CH,), jnp.int32),
            pltpu.VMEM((CH,), jnp.int32),
            pltpu.VMEM((CH, 32), jnp.float32),
            pltpu.VMEM((CH, 16), jnp.float32),
            pltpu.SemaphoreType.DMA,
            pltpu.SemaphoreType.DMA,
        ],
        compiler_params=pltpu.CompilerParams(use_tc_tiling_on_sc=False),
    )
    def k(t1h, t2h, srch, dsth, out1, idx1, idx2, buf1, buf2, sem1, sem2):
        w = lax.axis_index("s") * NC + lax.axis_index("c")

        def body(i, carry):
            row = w + i * NW

            @pl.when(row < ROWS)
            def _():
                pltpu.sync_copy(srch.at[row], idx1)
                pltpu.sync_copy(dsth.at[row], idx2)
                cp1 = pltpu.async_copy(t1h.at[idx1], buf1, sem1)
                cp2 = pltpu.async_copy(t2h.at[idx2], buf2, sem2)
                cp1.wait()
                cp2.wait()
                pltpu.sync_copy(buf1,
                                out1.at[pl.ds(row * CH, CH), pl.ds(0, 32)])
                pltpu.sync_copy(buf2,
                                out1.at[pl.ds(row * CH, CH), pl.ds(32, 16)])

            return carry

        lax.fori_loop(0, (ROWS + NW - 1) // NW, body, 0)

    return k(t1, t2, src2d, dst2d)


def _tc_msg(srcg, w1p, w2p, rh, rx, qa, qb, cq, prm):
    grid = N_EDGES // BE

    def body(prm_ref, w1_ref, w2_ref, rh_ref, rx_ref, qa_ref, qb_ref, cq_ref,
             s_ref, oa_ref, ob_ref):
        sg = s_ref[...]
        x = sg[:, 0:16]
        ev = sg[:, 32:40] - sg[:, 16:24]
        sq = ev * ev
        d2 = jnp.sum(sq, axis=1, keepdims=True)
        dist = jnp.sqrt(d2 + 1e-9)
        rinv = 1.0 / dist
        u9 = ev * rinv + prm_ref[2:3, 0:8]
        quad = _route(u9, qa_ref[...]) * _route(u9, qb_ref[...])
        s_sh = _route(quad, cq_ref[...])
        diff = (dist - prm_ref[0:1, :]) * prm_ref[1:2, :]
        t2 = diff * diff
        den = 1.0 - t2
        arg = -2.0 / den
        soft = (1.14136 * np.exp(2.0)) * jnp.exp(arg)
        valid = (t2 < 1.0) & (lax.broadcasted_iota(jnp.int32, (BE, 16), 1)
                              < NUM_BASIS)
        soft = jnp.where(valid, soft, 0.0)
        h = jax.nn.relu(jnp.dot(soft, w1_ref[...],
                                preferred_element_type=jnp.float32))
        g = (jnp.dot(h, rh_ref[...], preferred_element_type=jnp.float32)
             * jnp.dot(x, rx_ref[...], preferred_element_type=jnp.float32))
        m = jnp.dot(g, w2_ref[...], preferred_element_type=jnp.float32)
        msg = m * s_sh
        oa_ref[...] = msg[:, 0:128]
        ob_ref[:, 0:16] = msg[:, 128:144]

    small = lambda shp: pl.BlockSpec(shp, lambda i: (0, 0))
    return pl.pallas_call(
        body,
        grid=(grid,),
        in_specs=[
            small((8, 16)),
            small((16, 16)),
            small((256, OUT_DIM)),
            small((16, 256)),
            small((16, 256)),
            small((8, 16)),
            small((8, 16)),
            small((16, OUT_DIM)),
            pl.BlockSpec((BE, 128), lambda i: (i, 0)),
        ],
        out_specs=(pl.BlockSpec((BE, 128), lambda i: (i, 0)),
                   pl.BlockSpec((BE, 128), lambda i: (i, 0))),
        out_shape=(jax.ShapeDtypeStruct((N_EDGES, 128), jnp.float32),
                   jax.ShapeDtypeStruct((N_EDGES, 128), jnp.float32)),
    )(prm, w1p, w2p, rh, rx, qa, qb, cq, srcg)


def _sc_scatter(msga, msgb, dst2d):
    mesh = plsc.VectorSubcoreMesh(core_axis_name="c", subcore_axis_name="s")

    @functools.partial(
        pl.kernel,
        out_type=(jax.ShapeDtypeStruct((NC * N_NODES, 128), jnp.float32),
                  jax.ShapeDtypeStruct((NC * N_NODES, 16), jnp.float32)),
        mesh=mesh,
        scratch_types=[
            pltpu.VMEM((1, CH), jnp.int32),
            pltpu.VMEM((CH, 128), jnp.float32),
            pltpu.VMEM((CH, 16), jnp.float32),
            pltpu.VMEM_SHARED((N_NODES, 128), jnp.float32),
            pltpu.VMEM_SHARED((N_NODES, 16), jnp.float32),
        ],
        compiler_params=pltpu.CompilerParams(use_tc_tiling_on_sc=False),
    )
    def k(mah, mbh, dsth, outa, outb, idxb, bufa, bufb, acca, accb):
        c = lax.axis_index("c")
        s = lax.axis_index("s")
        zeros16 = jnp.zeros((16,), jnp.float32)

        def zrow(i, carry):
            def zcol(t, carry2):
                bufa[i, pl.ds(t * 16, 16)] = zeros16
                return carry2

            lax.fori_loop(0, 8, zcol, carry)
            bufb[i, :] = zeros16
            return carry

        lax.fori_loop(0, CH, zrow, 0)

        def zcp(t, carry):
            pltpu.sync_copy(bufa, acca.at[pl.ds(s * NPT + t * CH, CH)])
            pltpu.sync_copy(bufb, accb.at[pl.ds(s * NPT + t * CH, CH)])
            return carry

        lax.fori_loop(0, NPT // CH, zcp, 0)
        rem = NPT - (NPT // CH) * CH
        pltpu.sync_copy(bufa.at[pl.ds(0, rem)],
                        acca.at[pl.ds(s * NPT + NPT - rem, rem)])
        pltpu.sync_copy(bufb.at[pl.ds(0, rem)],
                        accb.at[pl.ds(s * NPT + NPT - rem, rem)])
        plsc.subcore_barrier()

        def body(i, carry):
            lrow = s + i * NS

            @pl.when(lrow < RPC)
            def _():
                row = c * RPC + lrow
                pltpu.sync_copy(dsth.at[row], idxb.at[0])
                pltpu.sync_copy(mah.at[pl.ds(row * CH, CH)], bufa)
                pltpu.sync_copy(mbh.at[pl.ds(row * CH, CH), pl.ds(0, 16)],
                                bufb)
                pltpu.sync_copy(bufa, acca.at[idxb.at[0]], add=True)
                pltpu.sync_copy(bufb, accb.at[idxb.at[0]], add=True)

            return carry

        lax.fori_loop(0, (RPC + NS - 1) // NS, body, 0)
        plsc.subcore_barrier()
        pltpu.sync_copy(acca.at[pl.ds(s * NPT, NPT)],
                        outa.at[pl.ds(c * N_NODES + s * NPT, NPT)])
        pltpu.sync_copy(accb.at[pl.ds(s * NPT, NPT)],
                        outb.at[pl.ds(c * N_NODES + s * NPT, NPT)])

    return k(msga, msgb, dst2d)


def _tc_add(parta, partb):
    def body(a_ref, b_ref, o_ref):
        o_ref[:, 0:128] = a_ref[0:N_NODES, :] + a_ref[N_NODES:2 * N_NODES, :]
        o_ref[:, 128:144] = b_ref[0:N_NODES, :] + b_ref[N_NODES:2 * N_NODES, :]

    return pl.pallas_call(
        body,
        out_shape=jax.ShapeDtypeStruct((N_NODES, OUT_DIM), jnp.float32),
    )(parta, partb)


def kernel(f_in, pos, edge_src, edge_dst, max_radius, W1, W2):
    f_in = f_in.astype(jnp.float32)
    pos = pos.astype(jnp.float32)
    t1 = jnp.concatenate([f_in, pos, jnp.zeros((N_NODES, 13), jnp.float32)], axis=1)
    t2 = jnp.concatenate([pos, jnp.zeros((N_NODES, 13), jnp.float32)], axis=1)
    src2d = edge_src.astype(jnp.int32).reshape(ROWS, CH)
    dst2d = edge_dst.astype(jnp.int32).reshape(ROWS, CH)

    w2r = W2.astype(jnp.float32).reshape(MUL, 3, MUL, MUL)
    w2kl = jnp.transpose(w2r, (0, 2, 1, 3)).reshape(MUL * MUL, 3 * MUL)
    w2p = (jnp.take(w2kl, jnp.asarray(_COLMAP), axis=1) * (1.0 / 64.0)
           * jnp.asarray(_COLCONST)[None, :])
    w1p = (jnp.zeros((16, 16), jnp.float32)
           .at[:NUM_BASIS].set(W1.astype(jnp.float32)) * np.sqrt(2.0))

    mr = jnp.asarray(max_radius, jnp.float32)
    step = mr / (NUM_BASIS + 1)
    vals = step * jnp.arange(1, NUM_BASIS + 1, dtype=jnp.float32)
    prm = (jnp.zeros((8, 16), jnp.float32)
           .at[0, :NUM_BASIS].set(vals)
           .at[1, :].set(1.0 / step)
           .at[2, 3].set(1.0))

    srcg = _sc_gather(t1, t2, src2d, dst2d)
    msga, msgb = _tc_msg(srcg, w1p, w2p,
                         jnp.asarray(_RH), jnp.asarray(_RX),
                         jnp.asarray(_QA, dtype=jnp.bfloat16),
                         jnp.asarray(_QB, dtype=jnp.bfloat16),
                         jnp.asarray(_CQ, dtype=jnp.bfloat16), prm)
    parta, partb = _sc_scatter(msga, msgb, dst2d)
    return _tc_add(parta, partb)


# speedup vs baseline: 2.0811x; 1.2210x over previous
import functools

import numpy as np
import jax
import jax.numpy as jnp
from jax import lax
from jax.experimental import pallas as pl
from jax.experimental.pallas import tpu as pltpu
from jax.experimental.pallas import tpu_sc as plsc

NUM_BASIS = 10
MUL = 16
N_NODES = 10000
N_EDGES = 160000
OUT_DIM = 144
NC, NS = 2, 16
NW = NC * NS
CH = 128
ROWS = N_EDGES // CH
RPC = ROWS // NC
NPT = N_NODES // NS
BE = 3200
BROWS = N_NODES * MUL // CH
NQ = 5
RQ = ROWS // NQ
EQ = RQ * CH
RPCQ = RQ // NC


def _static_mats():
    l = np.zeros(OUT_DIM, np.int32)
    v = np.zeros(OUT_DIM, np.int32)
    j = np.zeros(OUT_DIM, np.int32)
    for c in range(OUT_DIM):
        if c < 16:
            l[c], v[c], j[c] = 0, c, 0
        elif c < 64:
            l[c], v[c], j[c] = 1, (c - 16) // 3, (c - 16) % 3
        else:
            l[c], v[c], j[c] = 2, (c - 64) // 5, (c - 64) % 5
    colmap = l * MUL + v
    RH = np.zeros((MUL, MUL * MUL), np.float32)
    RX = np.zeros((MUL, MUL * MUL), np.float32)
    for k in range(MUL):
        for u in range(MUL):
            RH[k, k * MUL + u] = 1.0
            RX[u, k * MUL + u] = 1.0
    amap = [3, 0, 1, 2, 0, 0, 1, 1, 2, 0]
    bmap = [3, 3, 3, 3, 2, 1, 1, 2, 2, 0]
    A = np.zeros((8, 16), np.float32)
    B = np.zeros((8, 16), np.float32)
    for t in range(10):
        A[amap[t], t] = 1.0
        B[bmap[t], t] = 1.0
    r3, c2, r5 = np.sqrt(3.0), np.sqrt(15.0), np.sqrt(5.0)
    C = np.zeros((16, 9), np.float32)
    C[0, 0] = 1.0
    C[1, 1] = 1.0
    C[2, 2] = 1.0
    C[3, 3] = 1.0
    C[4, 4] = 1.0
    C[5, 5] = 1.0
    C[6, 6] = 3.0
    C[0, 6] = -1.0
    C[7, 7] = 1.0
    C[8, 8] = 1.0
    C[9, 8] = -1.0
    shconst = np.array([1.0, r3, r3, r3, c2, c2, r5 / 2.0, c2, c2 / 2.0],
                       np.float32)
    jg = np.where(l == 0, 0, np.where(l == 1, 1 + j, 4 + j))
    Q = np.zeros((9, OUT_DIM), np.float32)
    for c in range(OUT_DIM):
        Q[jg[c], c] = 1.0
    CQ = C @ Q
    colconst = shconst[jg]
    return colmap, colconst, RH, RX, A, B, CQ


_COLMAP, _COLCONST, _RH, _RX, _QA, _QB, _CQ = _static_mats()


def _split_bf16(x):
    hix = x.astype(jnp.bfloat16)
    lox = (x - hix.astype(jnp.float32)).astype(jnp.bfloat16)
    return hix, lox


def _route(x, sel):
    hix, lox = _split_bf16(x)
    return (jnp.dot(hix, sel, preferred_element_type=jnp.float32)
            + jnp.dot(lox, sel, preferred_element_type=jnp.float32))


def _sc_gather(t1, t2, src2d, dst2d):
    mesh = plsc.VectorSubcoreMesh(core_axis_name="c", subcore_axis_name="s")

    @functools.partial(
        pl.kernel,
        out_type=jax.ShapeDtypeStruct((EQ, 128), jnp.float32),
        mesh=mesh,
        scratch_types=[
            pltpu.VMEM((CH,), jnp.int32),
            pltpu.VMEM((CH,), jnp.int32),
            pltpu.VMEM((CH, 32), jnp.float32),
            pltpu.VMEM((CH, 16), jnp.float32),
            pltpu.SemaphoreType.DMA,
            pltpu.SemaphoreType.DMA,
        ],
        compiler_params=pltpu.CompilerParams(use_tc_tiling_on_sc=False),
    )
    def k(t1h, t2h, srch, dsth, out1, idx1, idx2, buf1, buf2, sem1, sem2):
        w = lax.axis_index("s") * NC + lax.axis_index("c")

        def body(i, carry):
            row = w + i * NW

            @pl.when(row < RQ)
            def _():
                pltpu.sync_copy(srch.at[row], idx1)
                pltpu.sync_copy(dsth.at[row], idx2)
                cp1 = pltpu.async_copy(t1h.at[idx1], buf1, sem1)
                cp2 = pltpu.async_copy(t2h.at[idx2], buf2, sem2)
                cp1.wait()
                cp2.wait()
                pltpu.sync_copy(buf1,
                                out1.at[pl.ds(row * CH, CH), pl.ds(0, 32)])
                pltpu.sync_copy(buf2,
                                out1.at[pl.ds(row * CH, CH), pl.ds(32, 16)])

            return carry

        lax.fori_loop(0, (RQ + NW - 1) // NW, body, 0)

    return k(t1, t2, src2d, dst2d)


def _tc_msg(srcg, w1p, w2p, rh, rx, qa, qb, cq, prm):
    grid = EQ // BE

    def body(prm_ref, w1_ref, w2_ref, rh_ref, rx_ref, qa_ref, qb_ref, cq_ref,
             s_ref, oa_ref, ob_ref):
        sg = s_ref[...]
        x = sg[:, 0:16]
        ev = sg[:, 32:40] - sg[:, 16:24]
        sq = ev * ev
        d2 = jnp.sum(sq, axis=1, keepdims=True)
        dist = jnp.sqrt(d2 + 1e-9)
        rinv = 1.0 / dist
        u9 = ev * rinv + prm_ref[2:3, 0:8]
        quad = _route(u9, qa_ref[...]) * _route(u9, qb_ref[...])
        s_sh = _route(quad, cq_ref[...])
        diff = (dist - prm_ref[0:1, :]) * prm_ref[1:2, :]
        t2 = diff * diff
        den = 1.0 - t2
        arg = -2.0 / den
        soft = (1.14136 * np.exp(2.0)) * jnp.exp(arg)
        valid = (t2 < 1.0) & (lax.broadcasted_iota(jnp.int32, (BE, 16), 1)
                              < NUM_BASIS)
        soft = jnp.where(valid, soft, 0.0)
        h = jax.nn.relu(jnp.dot(soft, w1_ref[...],
                                preferred_element_type=jnp.float32))
        g = (jnp.dot(h, rh_ref[...], preferred_element_type=jnp.float32)
             * jnp.dot(x, rx_ref[...], preferred_element_type=jnp.float32))
        m = jnp.dot(g, w2_ref[...], preferred_element_type=jnp.float32)
        msg = m * s_sh
        oa_ref[...] = msg[:, 0:128]
        ob_ref[:, 0:16] = msg[:, 128:144]

    small = lambda shp: pl.BlockSpec(shp, lambda i: (0, 0))
    return pl.pallas_call(
        body,
        grid=(grid,),
        in_specs=[
            small((8, 16)),
            small((16, 16)),
            small((256, OUT_DIM)),
            small((16, 256)),
            small((16, 256)),
            small((8, 16)),
            small((8, 16)),
            small((16, OUT_DIM)),
            pl.BlockSpec((BE, 128), lambda i: (i, 0)),
        ],
        out_specs=(pl.BlockSpec((BE, 128), lambda i: (i, 0)),
                   pl.BlockSpec((BE, 128), lambda i: (i, 0))),
        out_shape=(jax.ShapeDtypeStruct((EQ, 128), jnp.float32),
                   jax.ShapeDtypeStruct((EQ, 128), jnp.float32)),
    )(prm, w1p, w2p, rh, rx, qa, qb, cq, srcg)


def _sc_scatter(msga, msgb, dst2d):
    mesh = plsc.VectorSubcoreMesh(core_axis_name="c", subcore_axis_name="s")

    @functools.partial(
        pl.kernel,
        out_type=(jax.ShapeDtypeStruct((NC * N_NODES, 128), jnp.float32),
                  jax.ShapeDtypeStruct((NC * N_NODES, 16), jnp.float32)),
        mesh=mesh,
        scratch_types=[
            pltpu.VMEM((1, CH), jnp.int32),
            pltpu.VMEM((CH, 128), jnp.float32),
            pltpu.VMEM((CH, 16), jnp.float32),
            pltpu.VMEM_SHARED((N_NODES, 128), jnp.float32),
            pltpu.VMEM_SHARED((N_NODES, 16), jnp.float32),
        ],
        compiler_params=pltpu.CompilerParams(use_tc_tiling_on_sc=False),
    )
    def k(mah, mbh, dsth, outa, outb, idxb, bufa, bufb, acca, accb):
        c = lax.axis_index("c")
        s = lax.axis_index("s")
        zeros16 = jnp.zeros((16,), jnp.float32)

        def zrow(i, carry):
            def zcol(t, carry2):
                bufa[i, pl.ds(t * 16, 16)] = zeros16
                return carry2

            lax.fori_loop(0, 8, zcol, carry)
            bufb[i, :] = zeros16
            return carry

        lax.fori_loop(0, CH, zrow, 0)

        def zcp(t, carry):
            pltpu.sync_copy(bufa, acca.at[pl.ds(s * NPT + t * CH, CH)])
            pltpu.sync_copy(bufb, accb.at[pl.ds(s * NPT + t * CH, CH)])
            return carry

        lax.fori_loop(0, NPT // CH, zcp, 0)
        rem = NPT - (NPT // CH) * CH
        pltpu.sync_copy(bufa.at[pl.ds(0, rem)],
                        acca.at[pl.ds(s * NPT + NPT - rem, rem)])
        pltpu.sync_copy(bufb.at[pl.ds(0, rem)],
                        accb.at[pl.ds(s * NPT + NPT - rem, rem)])
        plsc.subcore_barrier()

        def body(i, carry):
            lrow = s + i * NS

            @pl.when(lrow < RPCQ)
            def _():
                row = c * RPCQ + lrow
                pltpu.sync_copy(dsth.at[row], idxb.at[0])
                pltpu.sync_copy(mah.at[pl.ds(row * CH, CH)], bufa)
                pltpu.sync_copy(mbh.at[pl.ds(row * CH, CH), pl.ds(0, 16)],
                                bufb)
                pltpu.sync_copy(bufa, acca.at[idxb.at[0]], add=True)
                pltpu.sync_copy(bufb, accb.at[idxb.at[0]], add=True)

            return carry

        lax.fori_loop(0, (RPCQ + NS - 1) // NS, body, 0)
        plsc.subcore_barrier()
        pltpu.sync_copy(acca.at[pl.ds(s * NPT, NPT)],
                        outa.at[pl.ds(c * N_NODES + s * NPT, NPT)])
        pltpu.sync_copy(accb.at[pl.ds(s * NPT, NPT)],
                        outb.at[pl.ds(c * N_NODES + s * NPT, NPT)])

    return k(msga, msgb, dst2d)


def _tc_add(partas, partbs):
    bn = 2000
    nb = N_NODES // bn

    def body(*refs):
        o_ref = refs[-1]
        arefs = refs[:2 * NQ]
        brefs = refs[2 * NQ:4 * NQ]
        a = arefs[0][...]
        for r in arefs[1:]:
            a = a + r[...]
        b = brefs[0][...]
        for r in brefs[1:]:
            b = b + r[...]
        o_ref[:, 0:128] = a
        o_ref[:, 128:144] = b

    in_specs = []
    args = []
    for p in partas:
        for off in (0, nb):
            in_specs.append(pl.BlockSpec((bn, 128), lambda i, o=off: (i + o, 0)))
            args.append(p)
    for p in partbs:
        for off in (0, nb):
            in_specs.append(pl.BlockSpec((bn, 16), lambda i, o=off: (i + o, 0)))
            args.append(p)
    return pl.pallas_call(
        body,
        grid=(nb,),
        in_specs=in_specs,
        out_specs=pl.BlockSpec((bn, OUT_DIM), lambda i: (i, 0)),
        out_shape=jax.ShapeDtypeStruct((N_NODES, OUT_DIM), jnp.float32),
    )(*args)


def kernel(f_in, pos, edge_src, edge_dst, max_radius, W1, W2):
    f_in = f_in.astype(jnp.float32)
    pos = pos.astype(jnp.float32)
    t1 = jnp.concatenate([f_in, pos, jnp.zeros((N_NODES, 13), jnp.float32)], axis=1)
    t2 = jnp.concatenate([pos, jnp.zeros((N_NODES, 13), jnp.float32)], axis=1)
    src2d = edge_src.astype(jnp.int32).reshape(ROWS, CH)
    dst2d = edge_dst.astype(jnp.int32).reshape(ROWS, CH)

    w2r = W2.astype(jnp.float32).reshape(MUL, 3, MUL, MUL)
    w2kl = jnp.transpose(w2r, (0, 2, 1, 3)).reshape(MUL * MUL, 3 * MUL)
    w2p = (jnp.take(w2kl, jnp.asarray(_COLMAP), axis=1) * (1.0 / 64.0)
           * jnp.asarray(_COLCONST)[None, :])
    w1p = (jnp.zeros((16, 16), jnp.float32)
           .at[:NUM_BASIS].set(W1.astype(jnp.float32)) * np.sqrt(2.0))

    mr = jnp.asarray(max_radius, jnp.float32)
    step = mr / (NUM_BASIS + 1)
    vals = step * jnp.arange(1, NUM_BASIS + 1, dtype=jnp.float32)
    prm = (jnp.zeros((8, 16), jnp.float32)
           .at[0, :NUM_BASIS].set(vals)
           .at[1, :].set(1.0 / step)
           .at[2, 3].set(1.0))

    rh = jnp.asarray(_RH)
    rx = jnp.asarray(_RX)
    qa = jnp.asarray(_QA, dtype=jnp.bfloat16)
    qb = jnp.asarray(_QB, dtype=jnp.bfloat16)
    cq = jnp.asarray(_CQ, dtype=jnp.bfloat16)

    partas, partbs = [], []
    for q in range(NQ):
        src_q = src2d[q * RQ:(q + 1) * RQ]
        dst_q = dst2d[q * RQ:(q + 1) * RQ]
        srcg = _sc_gather(t1, t2, src_q, dst_q)
        msga, msgb = _tc_msg(srcg, w1p, w2p, rh, rx, qa, qb, cq, prm)
        parta, partb = _sc_scatter(msga, msgb, dst_q)
        partas.append(parta)
        partbs.append(partb)
    return _tc_add(partas, partbs)
